# baseline (speedup 1.0000x reference)
import functools

import jax
import jax.numpy as jnp
from jax import lax
from jax.experimental import pallas as pl
from jax.experimental.pallas import tpu as pltpu
from jax.experimental.pallas import tpu_sc as plsc

_L = 512
_D = 128
_WIN_LO = _D * _D - 1 - (_L - 1)
_NWIN = 2 * _L

_BI = 32

_NC = 2
_NS = 16
_NW = _NC * _NS
_RPW = _NWIN // _NW
_LANES = 16


def _win_gather_kernel(table_hbm, win_hbm, idx_v, rows_v, sem):
    wid = lax.axis_index("s") * _NC + lax.axis_index("c")
    base = wid * _RPW
    for c in range(_RPW // _LANES):
        idx_v[pl.ds(c * _LANES, _LANES)] = (
            lax.iota(jnp.int32, _LANES) + (_WIN_LO + base + c * _LANES)
        )
    pltpu.async_copy(table_hbm.at[idx_v], rows_v, sem).wait()
    pltpu.sync_copy(rows_v, win_hbm.at[pl.ds(base, _RPW)])


def _sc_window(table):
    mesh = plsc.VectorSubcoreMesh(core_axis_name="c", subcore_axis_name="s")
    k = functools.partial(
        pl.kernel,
        mesh=mesh,
        out_type=jax.ShapeDtypeStruct((_NWIN, _D), jnp.float32),
        scratch_types=[
            pltpu.VMEM((_RPW,), jnp.int32),
            pltpu.VMEM((_RPW, _D), jnp.float32),
            pltpu.SemaphoreType.DMA,
        ],
    )(_win_gather_kernel)
    return k(table)


def _add_kernel(win_ref, x_ref, o_ref):
    ib = pl.program_id(1)
    base = _L - 1 - ib * _BI
    for li in range(_BI):
        shifted = win_ref[pl.ds(base - li, _L), :]
        o_ref[0, li] = x_ref[0, li] + shifted


def kernel(x, table):
    win = _sc_window(table)
    grid = (x.shape[0], _L // _BI)
    return pl.pallas_call(
        _add_kernel,
        grid=grid,
        in_specs=[
            pl.BlockSpec((_NWIN, _D), lambda b, i: (0, 0)),
            pl.BlockSpec((1, _BI, _L, _D), lambda b, i: (b, i, 0, 0)),
        ],
        out_specs=pl.BlockSpec((1, _BI, _L, _D), lambda b, i: (b, i, 0, 0)),
        out_shape=jax.ShapeDtypeStruct(x.shape, x.dtype),
        compiler_params=pltpu.CompilerParams(
            dimension_semantics=("parallel", "parallel"),
        ),
    )(win, x)
